# baseline (speedup 1.0000x reference)
import functools

import jax
import jax.numpy as jnp
from jax import lax
from jax.experimental import pallas as pl
from jax.experimental.pallas import tpu as pltpu
from jax.experimental.pallas import tpu_sc as plsc

HOPS = 3
EMB = 128
NEG_BIG = -1e30


def _ec(j):
  return pl.ds(16 * j, 16)


def _sc_body(nc, bpw, m_slots, t_width, ctx_hbm, c_hbm, out_hbm,
             idx_v, rows_a, rows_b, s_v, attn_ref, out_v, sem_a, sem_b):
  njc = EMB // 16
  nrows = m_slots * t_width
  wid = lax.axis_index("s") * nc + lax.axis_index("c")
  base = wid * bpw

  pltpu.sync_copy(ctx_hbm.at[pl.ds(base * 3 * nrows, bpw * 3 * nrows)], idx_v)

  lanes = lax.iota(jnp.int32, 16)
  nvp = (m_slots + 15) // 16

  def _lane_sum(v):
    for sh in (8, 4, 2, 1):
      v = v + v.at[lanes ^ sh].get(mode="promise_in_bounds")
    return v

  def _lane_max(v):
    for sh in (8, 4, 2, 1):
      v = jnp.maximum(v, v.at[lanes ^ sh].get(mode="promise_in_bounds"))
    return v

  def softmax_to_attn(v):
    mx = v[0]
    for k in range(1, nvp):
      mx = jnp.maximum(mx, v[k])
    mx = _lane_max(mx)
    e = [jnp.exp(vk - mx) for vk in v]
    s = e[0]
    for k in range(1, nvp):
      s = s + e[k]
    inv = 1.0 / _lane_sum(s)
    for k in range(nvp):
      attn_ref[pl.ds(16 * k, 16)] = e[k] * inv

  def p_loop(q):
    def body(m, pvecs):
      acc = s_v[m, _ec(0)] * q[0]
      for j in range(1, njc):
        acc = acc + s_v[m, _ec(j)] * q[j]
      pm = _lane_sum(acc)
      lane_upd = lanes == (m % 16)
      k_tgt = m // 16
      out = []
      for k in range(nvp):
        upd = jnp.where(lane_upd, pm, pvecs[k])
        out.append(jnp.where(k_tgt == k, upd, pvecs[k]))
      return tuple(out)
    init = tuple(jnp.full((16,), NEG_BIG, jnp.float32) for _ in range(nvp))
    return lax.fori_loop(0, m_slots, body, init, unroll=2)

  bufs = (rows_a, rows_b)
  sems = (sem_a, sem_b)

  def start(bl, t, p):
    off = bl * 3 * nrows + t * nrows
    pltpu.async_copy(c_hbm.at[idx_v.at[pl.ds(off, nrows)]], bufs[p], sems[p])

  def wait(p):
    pltpu.make_async_copy(c_hbm.at[pl.ds(0, nrows)], bufs[p], sems[p]).wait()

  def mat1_fn(rows_v):
    def mat1(m, sums):
      accs = []
      for j in range(njc):
        a = (rows_v[4 * m, _ec(j)] + rows_v[4 * m + 1, _ec(j)] +
             rows_v[4 * m + 2, _ec(j)] + rows_v[4 * m + 3, _ec(j)])
        s_v[m, _ec(j)] = a
        accs.append(sums[j] + a)
      return tuple(accs)
    zeros = tuple(jnp.zeros((16,), jnp.float32) for _ in range(njc))
    sums = lax.fori_loop(0, m_slots, mat1, zeros, unroll=5)
    return tuple(s * (1.0 / m_slots) for s in sums)

  def mat2_fn(rows_v, q):
    def mat2(m, qacc):
      a = attn_ref[pl.ds(m, 16)][0]
      accs = []
      for j in range(njc):
        r = (rows_v[4 * m, _ec(j)] + rows_v[4 * m + 1, _ec(j)] +
             rows_v[4 * m + 2, _ec(j)] + rows_v[4 * m + 3, _ec(j)])
        s_v[m, _ec(j)] = r
        accs.append(qacc[j] + r * a)
      return tuple(accs)
    return lax.fori_loop(0, m_slots, mat2, q, unroll=5)

  def o2_fn(rows_v, q):
    def o2(m, qacc):
      a = attn_ref[pl.ds(m, 16)][0]
      accs = list(qacc)
      for j in range(njc):
        r = (rows_v[4 * m, _ec(j)] + rows_v[4 * m + 1, _ec(j)] +
             rows_v[4 * m + 2, _ec(j)] + rows_v[4 * m + 3, _ec(j)])
        accs[j] = accs[j] + r * a
      return tuple(accs)
    return lax.fori_loop(0, m_slots, o2, q, unroll=5)

  def chain(bl, p0, next_b):
    p1, p2 = 1 - p0, p0
    wait(p0)
    start(bl, 1, p1)
    q = mat1_fn(bufs[p0])
    softmax_to_attn(p_loop(q))
    wait(p1)
    start(bl, 2, p2)
    q = mat2_fn(bufs[p1], q)
    softmax_to_attn(p_loop(q))
    wait(p2)
    start(next_b, 0, 1 - p2)
    q = o2_fn(bufs[p2], q)
    for j in range(njc):
      out_v[bl, _ec(j)] = q[j]

  start(0, 0, 0)
  def b_body(i, carry):
    b0 = 2 * i
    chain(b0, 0, b0 + 1)
    chain(b0 + 1, 1, jnp.minimum(b0 + 2, bpw - 1))
    return carry
  lax.fori_loop(0, bpw // 2, b_body, 0)
  wait(0)
  pltpu.sync_copy(out_v, out_hbm.at[pl.ds(base, bpw)])


@jax.jit
def kernel(context, C):
  B, M, T = context.shape
  nwords = C.shape[1]
  flat = context.reshape(B, 1, M * T)
  offs = (jnp.arange(1, HOPS + 1, dtype=jnp.int32) * nwords).reshape(1, HOPS, 1)
  idx3 = (flat + offs).reshape(-1)
  c_flat = C.reshape((HOPS + 1) * nwords, EMB)

  info = plsc.get_sparse_core_info()
  nc, ns = info.num_cores, info.num_subcores
  nw = nc * ns
  bpw = B // nw
  nrows = M * T

  mesh = plsc.VectorSubcoreMesh(core_axis_name="c", subcore_axis_name="s")
  body = functools.partial(_sc_body, nc, bpw, M, T)
  return pl.kernel(
      body,
      out_type=jax.ShapeDtypeStruct((B, EMB), jnp.float32),
      mesh=mesh,
      scratch_types=[
          pltpu.VMEM((bpw * HOPS * nrows,), jnp.int32),
          pltpu.VMEM((nrows, EMB), jnp.float32),
          pltpu.VMEM((nrows, EMB), jnp.float32),
          pltpu.VMEM((M, EMB), jnp.float32),
          pltpu.VMEM((80,), jnp.float32),
          pltpu.VMEM((bpw, EMB), jnp.float32),
          pltpu.SemaphoreType.DMA,
          pltpu.SemaphoreType.DMA,
      ],
  )(idx3, c_flat)


# speedup vs baseline: 1.0986x; 1.0986x over previous
import functools

import jax
import jax.numpy as jnp
from jax import lax
from jax.experimental import pallas as pl
from jax.experimental.pallas import tpu as pltpu
from jax.experimental.pallas import tpu_sc as plsc

HOPS = 3
EMB = 128
NEG_BIG = -1e30


def _ec(j):
  return pl.ds(16 * j, 16)


def _sc_body(nc, bpw, m_slots, t_width, ctx_hbm, c_hbm, out_hbm,
             idx_v, rows_a, rows_b, rows_c, s_v, attn_ref, out_v,
             sem_a, sem_b, sem_c):
  njc = EMB // 16
  nrows = m_slots * t_width
  wid = lax.axis_index("s") * nc + lax.axis_index("c")
  base = wid * bpw

  pltpu.sync_copy(ctx_hbm.at[pl.ds(base * 3 * nrows, bpw * 3 * nrows)], idx_v)

  lanes = lax.iota(jnp.int32, 16)
  nvp = (m_slots + 15) // 16

  def _lane_sum(v):
    for sh in (8, 4, 2, 1):
      v = v + v.at[lanes ^ sh].get(mode="promise_in_bounds")
    return v

  def _lane_max(v):
    for sh in (8, 4, 2, 1):
      v = jnp.maximum(v, v.at[lanes ^ sh].get(mode="promise_in_bounds"))
    return v

  def softmax_to_attn(v):
    mx = v[0]
    for k in range(1, nvp):
      mx = jnp.maximum(mx, v[k])
    mx = _lane_max(mx)
    e = [jnp.exp(vk - mx) for vk in v]
    s = e[0]
    for k in range(1, nvp):
      s = s + e[k]
    inv = 1.0 / _lane_sum(s)
    for k in range(nvp):
      attn_ref[pl.ds(16 * k, 16)] = e[k] * inv

  def p_loop(q):
    def body(m, pvecs):
      acc = s_v[m, _ec(0)] * q[0]
      for j in range(1, njc):
        acc = acc + s_v[m, _ec(j)] * q[j]
      pm = _lane_sum(acc)
      lane_upd = lanes == (m % 16)
      k_tgt = m // 16
      out = []
      for k in range(nvp):
        upd = jnp.where(lane_upd, pm, pvecs[k])
        out.append(jnp.where(k_tgt == k, upd, pvecs[k]))
      return tuple(out)
    init = tuple(jnp.full((16,), NEG_BIG, jnp.float32) for _ in range(nvp))
    return lax.fori_loop(0, m_slots, body, init, unroll=2)

  bufs = (rows_a, rows_b, rows_c)
  sems = (sem_a, sem_b, sem_c)

  def start(bl, t, p):
    off = bl * 3 * nrows + t * nrows
    pltpu.async_copy(c_hbm.at[idx_v.at[pl.ds(off, nrows)]], bufs[p], sems[p])

  def wait(p):
    pltpu.make_async_copy(c_hbm.at[pl.ds(0, nrows)], bufs[p], sems[p]).wait()

  def mat1_fn(rows_v):
    def mat1(m, sums):
      accs = []
      for j in range(njc):
        a = (rows_v[4 * m, _ec(j)] + rows_v[4 * m + 1, _ec(j)] +
             rows_v[4 * m + 2, _ec(j)] + rows_v[4 * m + 3, _ec(j)])
        s_v[m, _ec(j)] = a
        accs.append(sums[j] + a)
      return tuple(accs)
    zeros = tuple(jnp.zeros((16,), jnp.float32) for _ in range(njc))
    sums = lax.fori_loop(0, m_slots, mat1, zeros, unroll=5)
    return tuple(s * (1.0 / m_slots) for s in sums)

  def mat2_fn(rows_v, q):
    def mat2(m, qacc):
      a = attn_ref[pl.ds(m, 16)][0]
      accs = []
      for j in range(njc):
        r = (rows_v[4 * m, _ec(j)] + rows_v[4 * m + 1, _ec(j)] +
             rows_v[4 * m + 2, _ec(j)] + rows_v[4 * m + 3, _ec(j)])
        s_v[m, _ec(j)] = r
        accs.append(qacc[j] + r * a)
      return tuple(accs)
    return lax.fori_loop(0, m_slots, mat2, q, unroll=5)

  def o2_fn(rows_v, q):
    def o2(m, qacc):
      a = attn_ref[pl.ds(m, 16)][0]
      accs = list(qacc)
      for j in range(njc):
        r = (rows_v[4 * m, _ec(j)] + rows_v[4 * m + 1, _ec(j)] +
             rows_v[4 * m + 2, _ec(j)] + rows_v[4 * m + 3, _ec(j)])
        accs[j] = accs[j] + r * a
      return tuple(accs)
    return lax.fori_loop(0, m_slots, o2, q, unroll=5)

  start(0, 0, 0)
  start(0, 1, 1)

  def b_body(bl, carry):
    nb = jnp.minimum(bl + 1, bpw - 1)
    wait(0)
    start(bl, 2, 2)
    q = mat1_fn(bufs[0])
    softmax_to_attn(p_loop(q))
    wait(1)
    start(nb, 0, 0)
    q = mat2_fn(bufs[1], q)
    softmax_to_attn(p_loop(q))
    wait(2)
    start(nb, 1, 1)
    q = o2_fn(bufs[2], q)
    for j in range(njc):
      out_v[bl, _ec(j)] = q[j]
    return carry

  lax.fori_loop(0, bpw, b_body, 0)
  wait(0)
  wait(1)
  pltpu.sync_copy(out_v, out_hbm.at[pl.ds(base, bpw)])


@jax.jit
def kernel(context, C):
  B, M, T = context.shape
  nwords = C.shape[1]
  flat = context.reshape(B, 1, M * T)
  offs = (jnp.arange(1, HOPS + 1, dtype=jnp.int32) * nwords).reshape(1, HOPS, 1)
  idx3 = (flat + offs).reshape(-1)
  c_flat = C.reshape((HOPS + 1) * nwords, EMB)

  info = plsc.get_sparse_core_info()
  nc, ns = info.num_cores, info.num_subcores
  nw = nc * ns
  bpw = B // nw
  nrows = M * T

  mesh = plsc.VectorSubcoreMesh(core_axis_name="c", subcore_axis_name="s")
  body = functools.partial(_sc_body, nc, bpw, M, T)
  return pl.kernel(
      body,
      out_type=jax.ShapeDtypeStruct((B, EMB), jnp.float32),
      mesh=mesh,
      scratch_types=[
          pltpu.VMEM((bpw * HOPS * nrows,), jnp.int32),
          pltpu.VMEM((nrows, EMB), jnp.float32),
          pltpu.VMEM((nrows, EMB), jnp.float32),
          pltpu.VMEM((nrows, EMB), jnp.float32),
          pltpu.VMEM((M, EMB), jnp.float32),
          pltpu.VMEM((80,), jnp.float32),
          pltpu.VMEM((bpw, EMB), jnp.float32),
          pltpu.SemaphoreType.DMA,
          pltpu.SemaphoreType.DMA,
          pltpu.SemaphoreType.DMA,
      ],
  )(idx3, c_flat)


# speedup vs baseline: 1.1092x; 1.0096x over previous
import functools

import jax
import jax.numpy as jnp
from jax import lax
from jax.experimental import pallas as pl
from jax.experimental.pallas import tpu as pltpu
from jax.experimental.pallas import tpu_sc as plsc

HOPS = 3
EMB = 128
NEG_BIG = -1e30


def _ec(j):
  return pl.ds(16 * j, 16)


def _sc_body(nc, bpw, m_slots, t_width, ctx_hbm, c_hbm, out_hbm,
             idx_v, rows_a, rows_b, rows_c, s_v, attn_ref, out_v,
             sem_a, sem_b, sem_c):
  njc = EMB // 16
  nrows = m_slots * t_width
  wid = lax.axis_index("s") * nc + lax.axis_index("c")
  base = wid * bpw

  pltpu.sync_copy(ctx_hbm.at[pl.ds(base * 3 * nrows, bpw * 3 * nrows)], idx_v)

  lanes = lax.iota(jnp.int32, 16)
  nvp = (m_slots + 15) // 16

  def _lane_sum(v):
    for sh in (8, 4, 2, 1):
      v = v + v.at[lanes ^ sh].get(mode="promise_in_bounds")
    return v

  def _lane_max(v):
    for sh in (8, 4, 2, 1):
      v = jnp.maximum(v, v.at[lanes ^ sh].get(mode="promise_in_bounds"))
    return v

  def softmax_to_attn(v):
    mx = v[0]
    for k in range(1, nvp):
      mx = jnp.maximum(mx, v[k])
    mx = _lane_max(mx)
    e = [jnp.exp(vk - mx) for vk in v]
    s = e[0]
    for k in range(1, nvp):
      s = s + e[k]
    inv = 1.0 / _lane_sum(s)
    for k in range(nvp):
      attn_ref[pl.ds(16 * k, 16)] = e[k] * inv

  def p_loop(q):
    def body(m, pvecs):
      acc = s_v[m, _ec(0)] * q[0]
      for j in range(1, njc):
        acc = acc + s_v[m, _ec(j)] * q[j]
      pm = _lane_sum(acc)
      lane_upd = lanes == (m % 16)
      k_tgt = m // 16
      out = []
      for k in range(nvp):
        upd = jnp.where(lane_upd, pm, pvecs[k])
        out.append(jnp.where(k_tgt == k, upd, pvecs[k]))
      return tuple(out)
    init = tuple(jnp.full((16,), NEG_BIG, jnp.float32) for _ in range(nvp))
    return plsc.parallel_loop(0, m_slots, carry=init, unroll=2)(body)

  bufs = (rows_a, rows_b, rows_c)
  sems = (sem_a, sem_b, sem_c)

  def start(bl, t, p):
    off = bl * 3 * nrows + t * nrows
    pltpu.async_copy(c_hbm.at[idx_v.at[pl.ds(off, nrows)]], bufs[p], sems[p])

  def wait(p):
    pltpu.make_async_copy(c_hbm.at[pl.ds(0, nrows)], bufs[p], sems[p]).wait()

  def mat1_fn(rows_v):
    def mat1(m, sums):
      accs = []
      for j in range(njc):
        a = (rows_v[4 * m, _ec(j)] + rows_v[4 * m + 1, _ec(j)] +
             rows_v[4 * m + 2, _ec(j)] + rows_v[4 * m + 3, _ec(j)])
        s_v[m, _ec(j)] = a
        accs.append(sums[j] + a)
      return tuple(accs)
    zeros = tuple(jnp.zeros((16,), jnp.float32) for _ in range(njc))
    sums = plsc.parallel_loop(0, m_slots, carry=zeros, unroll=2)(mat1)
    return tuple(s * (1.0 / m_slots) for s in sums)

  def mat2_fn(rows_v, q):
    def mat2(m, qacc):
      a = attn_ref[pl.ds(m, 16)][0]
      accs = []
      for j in range(njc):
        r = (rows_v[4 * m, _ec(j)] + rows_v[4 * m + 1, _ec(j)] +
             rows_v[4 * m + 2, _ec(j)] + rows_v[4 * m + 3, _ec(j)])
        s_v[m, _ec(j)] = r
        accs.append(qacc[j] + r * a)
      return tuple(accs)
    return plsc.parallel_loop(0, m_slots, carry=q, unroll=2)(mat2)

  def o2_fn(rows_v, q):
    def o2(m, qacc):
      a = attn_ref[pl.ds(m, 16)][0]
      accs = list(qacc)
      for j in range(njc):
        r = (rows_v[4 * m, _ec(j)] + rows_v[4 * m + 1, _ec(j)] +
             rows_v[4 * m + 2, _ec(j)] + rows_v[4 * m + 3, _ec(j)])
        accs[j] = accs[j] + r * a
      return tuple(accs)
    return plsc.parallel_loop(0, m_slots, carry=q, unroll=2)(o2)

  start(0, 0, 0)
  start(0, 1, 1)

  def b_body(bl, carry):
    nb = jnp.minimum(bl + 1, bpw - 1)
    wait(0)
    start(bl, 2, 2)
    q = mat1_fn(bufs[0])
    softmax_to_attn(p_loop(q))
    wait(1)
    start(nb, 0, 0)
    q = mat2_fn(bufs[1], q)
    softmax_to_attn(p_loop(q))
    wait(2)
    start(nb, 1, 1)
    q = o2_fn(bufs[2], q)
    for j in range(njc):
      out_v[bl, _ec(j)] = q[j]
    return carry

  lax.fori_loop(0, bpw, b_body, 0)
  wait(0)
  wait(1)
  pltpu.sync_copy(out_v, out_hbm.at[pl.ds(base, bpw)])


@jax.jit
def kernel(context, C):
  B, M, T = context.shape
  nwords = C.shape[1]
  flat = context.reshape(B, 1, M * T)
  offs = (jnp.arange(1, HOPS + 1, dtype=jnp.int32) * nwords).reshape(1, HOPS, 1)
  idx3 = (flat + offs).reshape(-1)
  c_flat = C.reshape((HOPS + 1) * nwords, EMB)

  info = plsc.get_sparse_core_info()
  nc, ns = info.num_cores, info.num_subcores
  nw = nc * ns
  bpw = B // nw
  nrows = M * T

  mesh = plsc.VectorSubcoreMesh(core_axis_name="c", subcore_axis_name="s")
  body = functools.partial(_sc_body, nc, bpw, M, T)
  return pl.kernel(
      body,
      out_type=jax.ShapeDtypeStruct((B, EMB), jnp.float32),
      mesh=mesh,
      scratch_types=[
          pltpu.VMEM((bpw * HOPS * nrows,), jnp.int32),
          pltpu.VMEM((nrows, EMB), jnp.float32),
          pltpu.VMEM((nrows, EMB), jnp.float32),
          pltpu.VMEM((nrows, EMB), jnp.float32),
          pltpu.VMEM((M, EMB), jnp.float32),
          pltpu.VMEM((80,), jnp.float32),
          pltpu.VMEM((bpw, EMB), jnp.float32),
          pltpu.SemaphoreType.DMA,
          pltpu.SemaphoreType.DMA,
          pltpu.SemaphoreType.DMA,
      ],
  )(idx3, c_flat)
